# baseline (speedup 1.0000x reference)
import jax
import jax.numpy as jnp
from jax import lax
from jax.experimental import pallas as pl
from jax.experimental.pallas import tpu as pltpu
from jax.experimental.pallas import tpu_sc as plsc


_ROWS = 64
_COLS = 1_000_000
_BLOCK_C = 4096
_LANES = 128
_INT_MAX = 0x7FFFFFFF

_SC_CHUNK = 8192
_TC_BLOCKS = 190
_TC_COLS = _TC_BLOCKS * _BLOCK_C
_SC_NCH = 27
_SC_COLS = _SC_NCH * _SC_CHUNK
_TAIL_START = _TC_COLS + _SC_COLS
_TAIL = _COLS - _TAIL_START
_TAIL_CH = (_TAIL + _LANES - 1) // _LANES

_NC = 2
_NW = 32
_ROWS_PER_W = _ROWS // _NW

_K0 = 0
_K1 = 42
_K2 = _K0 ^ _K1 ^ 0x1BD11BDA

_ROT = ((13, 15, 26, 6), (17, 29, 16, 24))
_ADDS = ((1, 2, 1), (2, 0, 2), (0, 1, 3), (1, 2, 4), (2, 0, 5))

_CLN_IN = (0.9999998211860657, -0.5000069737434387, 0.3333568871021271,
           -0.24957780539989471, 0.19885361194610596, -0.17363153398036957,
           0.1633855551481247, -0.09913656115531921)
_CLN_OUT = (1.0000041723251343, -0.5000144839286804, 0.33298954367637634,
            -0.24898120760917664, 0.20673835277557373, -0.1877920776605606,
            0.11449082940816879)
_CP = (1.0, -0.5000001192092896, 0.33332598209381104, -0.2502099275588989,
       0.19696438312530518, -0.19059066474437714, 0.040718723088502884,
       -0.33426347374916077)
_LN2 = 0.6931471805599453
_SQRT2 = 1.4142135
_TINY = 1.1754943508222875e-38


def _threefry_bits(counter):
  ks = (jnp.uint32(_K0), jnp.uint32(_K1), jnp.uint32(_K2))
  x0 = jnp.zeros_like(counter) + ks[0]
  x1 = counter + ks[1]
  for g, (a, b, c) in enumerate(_ADDS):
    for r in _ROT[g % 2]:
      x0 = x0 + x1
      x1 = ((x1 << jnp.uint32(r)) | (x1 >> jnp.uint32(32 - r))) ^ x0
    x0 = x0 + ks[a]
    x1 = x1 + ks[b] + jnp.uint32(c)
  return x0 ^ x1


def _gumbel_from_bits(bits):
  tiny = jnp.float32(_TINY)
  float_bits = (bits >> jnp.uint32(9)) | jnp.uint32(0x3F800000)
  floats = lax.bitcast_convert_type(float_bits, jnp.float32) - jnp.float32(1.0)
  u = jnp.maximum(tiny, floats * (jnp.float32(1.0) - tiny) + tiny)
  return -jnp.log(-jnp.log(u))


def _block_gumbel(rows, cols, shape, col):
  r = lax.broadcasted_iota(jnp.uint32, shape, 0)
  counter = r * jnp.uint32(cols) + col
  return _gumbel_from_bits(_threefry_bits(counter))


def _chunk_reduce(vals, coli):
  m = jnp.max(vals, axis=1)
  idx = jnp.min(jnp.where(vals == m[:, None, :], coli, _INT_MAX), axis=1)
  return m, idx


def _tc_body(rows, cols, block_c):
  chunks = block_c // _LANES

  def body(x_ref, oval_ref, oidx_ref):
    i = pl.program_id(0)

    @pl.when(i == 0)
    def _init():
      oval_ref[...] = jnp.full((rows, _LANES), -jnp.inf, jnp.float32)
      oidx_ref[...] = jnp.zeros((rows, _LANES), jnp.int32)

    shape = (rows, chunks, _LANES)
    ch = lax.broadcasted_iota(jnp.uint32, shape, 1)
    ln = lax.broadcasted_iota(jnp.uint32, shape, 2)
    col = jnp.uint32(i) * jnp.uint32(block_c) + ch * jnp.uint32(_LANES) + ln
    vals = x_ref[...].reshape(shape) + _block_gumbel(rows, cols, shape, col)
    m, idx = _chunk_reduce(vals, col.astype(jnp.int32))

    sval = oval_ref[...]
    sidx = oidx_ref[...]
    better = (m > sval) | ((m == sval) & (idx < sidx))
    oval_ref[...] = jnp.where(better, m, sval)
    oidx_ref[...] = jnp.where(better, idx, sidx)

  return body


def _copy_body(x_ref, out_ref):
  out_ref[...] = x_ref[...].reshape(out_ref.shape)


def _merge_body(rows, cols, tail_start, tail_ch):
  def body(xt_ref, mval_ref, midx_ref, scv_ref, sci_ref, out_ref):
    shape = (rows, tail_ch, _LANES)
    ch = lax.broadcasted_iota(jnp.uint32, shape, 1)
    ln = lax.broadcasted_iota(jnp.uint32, shape, 2)
    col = jnp.uint32(tail_start) + ch * jnp.uint32(_LANES) + ln
    vals = xt_ref[...].reshape(shape) + _block_gumbel(rows, cols, shape, col)
    vals = jnp.where(col < jnp.uint32(cols), vals, -jnp.inf)
    tm, tidx = _chunk_reduce(vals, col.astype(jnp.int32))

    mv = mval_ref[...]
    mi = midx_ref[...]
    better = (tm > mv) | ((tm == mv) & (tidx < mi))
    v = jnp.where(better, tm, mv)
    ix = jnp.where(better, tidx, mi)
    tmax = jnp.max(v, axis=1)
    tcidx = jnp.min(jnp.where(v == tmax[:, None], ix, _INT_MAX), axis=1)

    sv = scv_ref[...]
    si = sci_ref[...]
    smax = jnp.max(sv, axis=1)
    sidx = jnp.min(jnp.where(sv == smax[:, None], si, _INT_MAX), axis=1)

    better2 = (smax > tmax) | ((smax == tmax) & (sidx < tcidx))
    first = jnp.where(better2, sidx, tcidx)
    out_ref[...] = first.reshape(1, rows)

  return body


def _sc_ln(y, coefs):
  b = lax.bitcast_convert_type(y, jnp.int32)
  e = lax.shift_right_logical(b, 23) - 127
  vb = (b & 0x7FFFFF) | 0x3F800000
  v = lax.bitcast_convert_type(vb, jnp.float32)
  big = v >= jnp.float32(_SQRT2)
  v = jnp.where(big, v * jnp.float32(0.5), v)
  e = jnp.where(big, e + 1, e)
  d = v - jnp.float32(1.0)
  acc = jnp.full_like(d, coefs[-1])
  for c in coefs[-2::-1]:
    acc = acc * d + jnp.float32(c)
  lnv = acc * d
  return e.astype(jnp.float32) * jnp.float32(_LN2) + lnv


def _sc_gumbel(bits):
  m = lax.shift_right_logical(bits, 9)
  mf = m.astype(jnp.float32)
  u = jnp.maximum(mf * jnp.float32(2.0 ** -23), jnp.float32(_TINY))
  w = (mf - jnp.float32(2.0 ** 23)) * jnp.float32(2.0 ** -23)
  accp = jnp.full_like(w, _CP[-1])
  for c in _CP[-2::-1]:
    accp = accp * w + jnp.float32(c)
  t_a = -(w * accp)
  t_b = -_sc_ln(u, _CLN_IN)
  t = jnp.where(m >= (3 << 21), t_a, t_b)
  return -_sc_ln(t, _CLN_OUT)


def _sc_threefry_bits_i32(counter):
  ks = (jnp.int32(_K0), jnp.int32(_K1), jnp.int32(_K2))
  x0 = jnp.zeros_like(counter) + ks[0]
  x1 = counter + ks[1]
  for g, (a, b, c) in enumerate(_ADDS):
    for r in _ROT[g % 2]:
      x0 = x0 + x1
      x1 = ((x1 << r) | lax.shift_right_logical(x1, 32 - r)) ^ x0
    x0 = x0 + ks[a]
    x1 = x1 + ks[b] + jnp.int32(c)
  return x0 ^ x1


def _sc_body(x1d_ref, outv_ref, outi_ref, buf_ref, bv16_ref, bi16_ref):
  wid = lax.axis_index("s") * _NC + lax.axis_index("c")
  grp = wid // 4
  q = wid % 4
  iota = lax.iota(jnp.int32, 16)

  for rr in range(_ROWS_PER_W):
    sr = 2 * q + rr
    row = 8 * grp + sr
    carry = (jnp.full((16,), -jnp.inf, jnp.float32),
             jnp.zeros((16,), jnp.int32))

    def chunk_body(j, c2, row=row, grp=grp, sr=sr):
      off = ((grp * _SC_NCH + j) * 8 + sr) * _SC_CHUNK
      pltpu.sync_copy(x1d_ref.at[pl.ds(off, _SC_CHUNK)], buf_ref)
      col0 = _TC_COLS + j * _SC_CHUNK
      base_flat = row * _COLS + col0

      def vec_body(v, c3):
        bv, bi = c3
        xv = buf_ref[pl.ds(v * 16, 16)]
        cnt = (base_flat + v * 16) + iota
        g = _sc_gumbel(_sc_threefry_bits_i32(cnt))
        val = xv + g
        ci = (col0 + v * 16) + iota
        pred = val > bv
        return (jnp.where(pred, val, bv), jnp.where(pred, ci, bi))

      return lax.fori_loop(0, _SC_CHUNK // 16, vec_body, c2, unroll=4)

    bv, bi = lax.fori_loop(0, _SC_NCH, chunk_body, carry)

    bv16_ref[...] = bv
    bi16_ref[...] = bi
    pltpu.sync_copy(bv16_ref, outv_ref.at[pl.ds(row * 16, 16)])
    pltpu.sync_copy(bi16_ref, outi_ref.at[pl.ds(row * 16, 16)])


def _sc_partials(x1d):
  run = pl.kernel(
      _sc_body,
      out_type=(jax.ShapeDtypeStruct((_ROWS * 16,), jnp.float32),
                jax.ShapeDtypeStruct((_ROWS * 16,), jnp.int32)),
      mesh=plsc.VectorSubcoreMesh(
          core_axis_name="c", subcore_axis_name="s", num_cores=_NC),
      scratch_types=(pltpu.VMEM((_SC_CHUNK,), jnp.float32),
                     pltpu.VMEM((16,), jnp.float32),
                     pltpu.VMEM((16,), jnp.int32)),
  )
  return run(x1d)


def _run(x, rows, cols, block_c):
  x_sc = pl.pallas_call(
      _copy_body,
      grid=(rows // 8, _SC_NCH),
      in_specs=[pl.BlockSpec((8, _SC_CHUNK),
                             lambda i, j: (i, _TC_COLS // _SC_CHUNK + j))],
      out_specs=pl.BlockSpec((8 * _SC_CHUNK,), lambda i, j: (i * _SC_NCH + j,)),
      out_shape=jax.ShapeDtypeStruct((rows * _SC_COLS,), jnp.float32),
  )(x)

  scv, sci = _sc_partials(x_sc)
  scv = scv.reshape(rows, 16)
  sci = sci.reshape(rows, 16)

  mval, midx = pl.pallas_call(
      _tc_body(rows, cols, block_c),
      grid=(_TC_BLOCKS,),
      in_specs=[pl.BlockSpec((rows, block_c), lambda i: (0, i))],
      out_specs=[pl.BlockSpec((rows, _LANES), lambda i: (0, 0)),
                 pl.BlockSpec((rows, _LANES), lambda i: (0, 0))],
      out_shape=[jax.ShapeDtypeStruct((rows, _LANES), jnp.float32),
                 jax.ShapeDtypeStruct((rows, _LANES), jnp.int32)],
  )(x)

  xt = x[:, _TAIL_START:]
  pad = _TAIL_CH * _LANES - _TAIL
  if pad:
    xt = jnp.concatenate(
        [xt, jnp.full((rows, pad), -jnp.inf, jnp.float32)], axis=1)

  out = pl.pallas_call(
      _merge_body(rows, cols, _TAIL_START, _TAIL_CH),
      in_specs=[pl.BlockSpec(xt.shape, lambda: (0, 0)),
                pl.BlockSpec((rows, _LANES), lambda: (0, 0)),
                pl.BlockSpec((rows, _LANES), lambda: (0, 0)),
                pl.BlockSpec((rows, 16), lambda: (0, 0)),
                pl.BlockSpec((rows, 16), lambda: (0, 0))],
      out_specs=pl.BlockSpec((1, rows), lambda: (0, 0)),
      out_shape=jax.ShapeDtypeStruct((1, rows), jnp.int32),
  )(xt, mval, midx, scv, sci)
  return out.reshape(rows)


@jax.jit
def kernel(x):
  return _run(x, _ROWS, _COLS, _BLOCK_C)


# speedup vs baseline: 1.0570x; 1.0570x over previous
import jax
import jax.numpy as jnp
from jax import lax
from jax.experimental import pallas as pl
from jax.experimental.pallas import tpu as pltpu
from jax.experimental.pallas import tpu_sc as plsc


_ROWS = 64
_COLS = 1_000_000
_BLOCK_C = 4096
_LANES = 128
_INT_MAX = 0x7FFFFFFF

_SC_CHUNK = 16384
_TC_BLOCKS = 192
_TC_COLS = _TC_BLOCKS * _BLOCK_C
_SC_NCH = 13
_SC_COLS = _SC_NCH * _SC_CHUNK
_TAIL_START = _TC_COLS + _SC_COLS
_TAIL = _COLS - _TAIL_START
_TAIL_CH = (_TAIL + _LANES - 1) // _LANES

_NC = 2
_NW = 32
_ROWS_PER_W = _ROWS // _NW

_K0 = 0
_K1 = 42
_K2 = _K0 ^ _K1 ^ 0x1BD11BDA

_ROT = ((13, 15, 26, 6), (17, 29, 16, 24))
_ADDS = ((1, 2, 1), (2, 0, 2), (0, 1, 3), (1, 2, 4), (2, 0, 5))

_CLN_IN = (0.9999998211860657, -0.5000069737434387, 0.3333568871021271,
           -0.24957780539989471, 0.19885361194610596, -0.17363153398036957,
           0.1633855551481247, -0.09913656115531921)
_CLN_OUT = (1.0000041723251343, -0.5000144839286804, 0.33298954367637634,
            -0.24898120760917664, 0.20673835277557373, -0.1877920776605606,
            0.11449082940816879)
_CP = (1.0, -0.5000001192092896, 0.33332598209381104, -0.2502099275588989,
       0.19696438312530518, -0.19059066474437714, 0.040718723088502884,
       -0.33426347374916077)
_LN2 = 0.6931471805599453
_SQRT2 = 1.4142135
_TINY = 1.1754943508222875e-38


def _threefry_bits(counter):
  ks = (jnp.uint32(_K0), jnp.uint32(_K1), jnp.uint32(_K2))
  x0 = jnp.zeros_like(counter) + ks[0]
  x1 = counter + ks[1]
  for g, (a, b, c) in enumerate(_ADDS):
    for r in _ROT[g % 2]:
      x0 = x0 + x1
      x1 = ((x1 << jnp.uint32(r)) | (x1 >> jnp.uint32(32 - r))) ^ x0
    x0 = x0 + ks[a]
    x1 = x1 + ks[b] + jnp.uint32(c)
  return x0 ^ x1


def _gumbel_from_bits(bits):
  tiny = jnp.float32(_TINY)
  float_bits = (bits >> jnp.uint32(9)) | jnp.uint32(0x3F800000)
  floats = lax.bitcast_convert_type(float_bits, jnp.float32) - jnp.float32(1.0)
  u = jnp.maximum(tiny, floats * (jnp.float32(1.0) - tiny) + tiny)
  return -jnp.log(-jnp.log(u))


def _block_gumbel(rows, cols, shape, col):
  r = lax.broadcasted_iota(jnp.uint32, shape, 0)
  counter = r * jnp.uint32(cols) + col
  return _gumbel_from_bits(_threefry_bits(counter))


def _chunk_reduce(vals, coli):
  m = jnp.max(vals, axis=1)
  idx = jnp.min(jnp.where(vals == m[:, None, :], coli, _INT_MAX), axis=1)
  return m, idx


def _tc_body(rows, cols, block_c):
  chunks = block_c // _LANES

  def body(x_ref, oval_ref, oidx_ref):
    i = pl.program_id(0)

    @pl.when(i == 0)
    def _init():
      oval_ref[...] = jnp.full((rows, _LANES), -jnp.inf, jnp.float32)
      oidx_ref[...] = jnp.zeros((rows, _LANES), jnp.int32)

    shape = (rows, chunks, _LANES)
    ch = lax.broadcasted_iota(jnp.uint32, shape, 1)
    ln = lax.broadcasted_iota(jnp.uint32, shape, 2)
    col = jnp.uint32(i) * jnp.uint32(block_c) + ch * jnp.uint32(_LANES) + ln
    vals = x_ref[...].reshape(shape) + _block_gumbel(rows, cols, shape, col)
    m, idx = _chunk_reduce(vals, col.astype(jnp.int32))

    sval = oval_ref[...]
    sidx = oidx_ref[...]
    better = (m > sval) | ((m == sval) & (idx < sidx))
    oval_ref[...] = jnp.where(better, m, sval)
    oidx_ref[...] = jnp.where(better, idx, sidx)

  return body


def _copy_body(x_ref, out_ref):
  out_ref[...] = x_ref[...].reshape(out_ref.shape)


def _merge_body(rows, cols, tail_start, tail_ch):
  def body(xt_ref, mval_ref, midx_ref, scv_ref, sci_ref, out_ref):
    shape = (rows, tail_ch, _LANES)
    ch = lax.broadcasted_iota(jnp.uint32, shape, 1)
    ln = lax.broadcasted_iota(jnp.uint32, shape, 2)
    col = jnp.uint32(tail_start) + ch * jnp.uint32(_LANES) + ln
    vals = xt_ref[...].reshape(shape) + _block_gumbel(rows, cols, shape, col)
    vals = jnp.where(col < jnp.uint32(cols), vals, -jnp.inf)
    tm, tidx = _chunk_reduce(vals, col.astype(jnp.int32))

    mv = mval_ref[...]
    mi = midx_ref[...]
    better = (tm > mv) | ((tm == mv) & (tidx < mi))
    v = jnp.where(better, tm, mv)
    ix = jnp.where(better, tidx, mi)
    tmax = jnp.max(v, axis=1)
    tcidx = jnp.min(jnp.where(v == tmax[:, None], ix, _INT_MAX), axis=1)

    sv = scv_ref[...]
    si = sci_ref[...]
    smax = jnp.max(sv, axis=1)
    sidx = jnp.min(jnp.where(sv == smax[:, None], si, _INT_MAX), axis=1)

    better2 = (smax > tmax) | ((smax == tmax) & (sidx < tcidx))
    first = jnp.where(better2, sidx, tcidx)
    out_ref[...] = first.reshape(1, rows)

  return body


def _sc_ln(y, coefs):
  b = lax.bitcast_convert_type(y, jnp.int32)
  e = lax.shift_right_logical(b, 23) - 127
  vb = (b & 0x7FFFFF) | 0x3F800000
  v = lax.bitcast_convert_type(vb, jnp.float32)
  big = v >= jnp.float32(_SQRT2)
  v = jnp.where(big, v * jnp.float32(0.5), v)
  e = jnp.where(big, e + 1, e)
  d = v - jnp.float32(1.0)
  acc = jnp.full_like(d, coefs[-1])
  for c in coefs[-2::-1]:
    acc = acc * d + jnp.float32(c)
  lnv = acc * d
  return e.astype(jnp.float32) * jnp.float32(_LN2) + lnv


def _sc_gumbel(bits):
  m = lax.shift_right_logical(bits, 9)
  mf = m.astype(jnp.float32)
  u = jnp.maximum(mf * jnp.float32(2.0 ** -23), jnp.float32(_TINY))
  w = (mf - jnp.float32(2.0 ** 23)) * jnp.float32(2.0 ** -23)
  accp = jnp.full_like(w, _CP[-1])
  for c in _CP[-2::-1]:
    accp = accp * w + jnp.float32(c)
  t_a = -(w * accp)
  t_b = -_sc_ln(u, _CLN_IN)
  t = jnp.where(m >= (3 << 21), t_a, t_b)
  return -_sc_ln(t, _CLN_OUT)


def _sc_threefry_bits_i32(counter):
  ks = (jnp.int32(_K0), jnp.int32(_K1), jnp.int32(_K2))
  x0 = jnp.zeros_like(counter) + ks[0]
  x1 = counter + ks[1]
  for g, (a, b, c) in enumerate(_ADDS):
    for r in _ROT[g % 2]:
      x0 = x0 + x1
      x1 = ((x1 << r) | lax.shift_right_logical(x1, 32 - r)) ^ x0
    x0 = x0 + ks[a]
    x1 = x1 + ks[b] + jnp.int32(c)
  return x0 ^ x1


def _sc_body(x1d_ref, outv_ref, outi_ref, buf_ref, bv16_ref, bi16_ref, sem):
  wid = lax.axis_index("s") * _NC + lax.axis_index("c")
  grp = wid // 4
  q = wid % 4
  iota = lax.iota(jnp.int32, 16)

  for rr in range(_ROWS_PER_W):
    sr = 2 * q + rr
    row = 8 * grp + sr
    carry = (jnp.full((16,), -jnp.inf, jnp.float32),
             jnp.zeros((16,), jnp.int32))

    def stage_off(j, grp=grp, sr=sr):
      return ((grp * _SC_NCH + j) * 8 + sr) * _SC_CHUNK

    pltpu.make_async_copy(x1d_ref.at[pl.ds(stage_off(0), _SC_CHUNK)],
                          buf_ref.at[pl.ds(0, _SC_CHUNK)], sem).start()

    def chunk_body(j, c2, row=row):
      par = (j & 1) * _SC_CHUNK
      pltpu.make_async_copy(x1d_ref.at[pl.ds(0, _SC_CHUNK)],
                            buf_ref.at[pl.ds(par, _SC_CHUNK)], sem).wait()

      @pl.when(j + 1 < _SC_NCH)
      def _prefetch():
        npar = ((j + 1) & 1) * _SC_CHUNK
        pltpu.make_async_copy(
            x1d_ref.at[pl.ds(stage_off(j + 1), _SC_CHUNK)],
            buf_ref.at[pl.ds(npar, _SC_CHUNK)], sem).start()

      col0 = _TC_COLS + j * _SC_CHUNK
      base_flat = row * _COLS + col0

      def vec_body(v, c3):
        bv, bi = c3
        xv = buf_ref[pl.ds(par + v * 16, 16)]
        cnt = (base_flat + v * 16) + iota
        g = _sc_gumbel(_sc_threefry_bits_i32(cnt))
        val = xv + g
        ci = (col0 + v * 16) + iota
        pred = val > bv
        return (jnp.where(pred, val, bv), jnp.where(pred, ci, bi))

      return lax.fori_loop(0, _SC_CHUNK // 16, vec_body, c2, unroll=4)

    bv, bi = lax.fori_loop(0, _SC_NCH, chunk_body, carry)

    bv16_ref[...] = bv
    bi16_ref[...] = bi
    pltpu.sync_copy(bv16_ref, outv_ref.at[pl.ds(row * 16, 16)])
    pltpu.sync_copy(bi16_ref, outi_ref.at[pl.ds(row * 16, 16)])


def _sc_partials(x1d):
  run = pl.kernel(
      _sc_body,
      out_type=(jax.ShapeDtypeStruct((_ROWS * 16,), jnp.float32),
                jax.ShapeDtypeStruct((_ROWS * 16,), jnp.int32)),
      mesh=plsc.VectorSubcoreMesh(
          core_axis_name="c", subcore_axis_name="s", num_cores=_NC),
      scratch_types=(pltpu.VMEM((2 * _SC_CHUNK,), jnp.float32),
                     pltpu.VMEM((16,), jnp.float32),
                     pltpu.VMEM((16,), jnp.int32),
                     pltpu.SemaphoreType.DMA),
  )
  return run(x1d)


def _run(x, rows, cols, block_c):
  x_sc = pl.pallas_call(
      _copy_body,
      grid=(rows // 8, _SC_NCH),
      in_specs=[pl.BlockSpec((8, _SC_CHUNK),
                             lambda i, j: (i, _TC_COLS // _SC_CHUNK + j))],
      out_specs=pl.BlockSpec((8 * _SC_CHUNK,), lambda i, j: (i * _SC_NCH + j,)),
      out_shape=jax.ShapeDtypeStruct((rows * _SC_COLS,), jnp.float32),
  )(x)

  scv, sci = _sc_partials(x_sc)
  scv = scv.reshape(rows, 16)
  sci = sci.reshape(rows, 16)

  mval, midx = pl.pallas_call(
      _tc_body(rows, cols, block_c),
      grid=(_TC_BLOCKS,),
      in_specs=[pl.BlockSpec((rows, block_c), lambda i: (0, i))],
      out_specs=[pl.BlockSpec((rows, _LANES), lambda i: (0, 0)),
                 pl.BlockSpec((rows, _LANES), lambda i: (0, 0))],
      out_shape=[jax.ShapeDtypeStruct((rows, _LANES), jnp.float32),
                 jax.ShapeDtypeStruct((rows, _LANES), jnp.int32)],
  )(x)

  xt = x[:, _TAIL_START:]
  pad = _TAIL_CH * _LANES - _TAIL
  if pad:
    xt = jnp.concatenate(
        [xt, jnp.full((rows, pad), -jnp.inf, jnp.float32)], axis=1)

  out = pl.pallas_call(
      _merge_body(rows, cols, _TAIL_START, _TAIL_CH),
      in_specs=[pl.BlockSpec(xt.shape, lambda: (0, 0)),
                pl.BlockSpec((rows, _LANES), lambda: (0, 0)),
                pl.BlockSpec((rows, _LANES), lambda: (0, 0)),
                pl.BlockSpec((rows, 16), lambda: (0, 0)),
                pl.BlockSpec((rows, 16), lambda: (0, 0))],
      out_specs=pl.BlockSpec((1, rows), lambda: (0, 0)),
      out_shape=jax.ShapeDtypeStruct((1, rows), jnp.int32),
  )(xt, mval, midx, scv, sci)
  return out.reshape(rows)


@jax.jit
def kernel(x):
  return _run(x, _ROWS, _COLS, _BLOCK_C)
